# baseline (speedup 1.0000x reference)
import functools

import jax
import jax.numpy as jnp
from jax import lax
from jax.experimental import pallas as pl
from jax.experimental.pallas import tpu as pltpu
from jax.experimental.pallas import tpu_sc as plsc

BATCH = 16384
EMBED = 64
LANES = 16

_info = plsc.get_sparse_core_info()
_NC = _info.num_cores
_NS = _info.num_subcores
_NW = _NC * _NS
_B_PER_W = BATCH // _NW
_CHUNK = 128
_NCHUNK = _B_PER_W // _CHUNK

_mesh = plsc.VectorSubcoreMesh(core_axis_name="c", subcore_axis_name="s")


@functools.partial(
    pl.kernel,
    mesh=_mesh,
    out_type=jax.ShapeDtypeStruct((BATCH, EMBED), jnp.float32),
    compiler_params=pltpu.CompilerParams(use_tc_tiling_on_sc=False),
    scratch_types=[
        pltpu.VMEM((_NCHUNK, _CHUNK), jnp.int32),
        pltpu.VMEM((_NCHUNK, _CHUNK), jnp.int32),
        pltpu.VMEM((_B_PER_W, EMBED), jnp.float32),
        pltpu.VMEM((_B_PER_W, EMBED), jnp.float32),
        pltpu.SemaphoreType.DMA,
    ],
)
def _gmf_sc(uidx_hbm, iidx_hbm, utab_hbm, itab_hbm, out_hbm,
            uidx_v, iidx_v, urows_v, irows_v, sem):
    wid = lax.axis_index("s") * _NC + lax.axis_index("c")
    base = wid * _B_PER_W

    pltpu.sync_copy(uidx_hbm.at[wid], uidx_v)
    pltpu.sync_copy(iidx_hbm.at[wid], iidx_v)

    copies = []
    for j in range(_NCHUNK):
        dst = urows_v.at[pl.ds(j * _CHUNK, _CHUNK)]
        copies.append(pltpu.async_copy(utab_hbm.at[uidx_v.at[j]], dst, sem))
    for j in range(_NCHUNK):
        dst = irows_v.at[pl.ds(j * _CHUNK, _CHUNK)]
        copies.append(pltpu.async_copy(itab_hbm.at[iidx_v.at[j]], dst, sem))
    for c in copies:
        c.wait()

    def row_body(r, _):
        for cbase in range(0, EMBED, LANES):
            sl = pl.ds(cbase, LANES)
            urows_v[r, sl] = urows_v[r, sl] * irows_v[r, sl]
        return 0

    lax.fori_loop(0, _B_PER_W, row_body, 0)

    pltpu.sync_copy(urows_v, out_hbm.at[pl.ds(base, _B_PER_W)])


def kernel(user_indices, item_indices, user_table, item_table):
    uidx = user_indices.astype(jnp.int32).reshape(_NW, _NCHUNK, _CHUNK)
    iidx = item_indices.astype(jnp.int32).reshape(_NW, _NCHUNK, _CHUNK)
    return _gmf_sc(uidx, iidx, user_table, item_table)


# speedup vs baseline: 1.0006x; 1.0006x over previous
import functools

import jax
import jax.numpy as jnp
from jax import lax
from jax.experimental import pallas as pl
from jax.experimental.pallas import tpu as pltpu
from jax.experimental.pallas import tpu_sc as plsc

BATCH = 16384
EMBED = 64
LANES = 16

_info = plsc.get_sparse_core_info()
_NC = _info.num_cores
_NS = _info.num_subcores
_NW = _NC * _NS
_B_PER_W = BATCH // _NW
_CHUNK = 128
_NCHUNK = _B_PER_W // _CHUNK

_mesh = plsc.VectorSubcoreMesh(core_axis_name="c", subcore_axis_name="s")


@functools.partial(
    pl.kernel,
    mesh=_mesh,
    out_type=jax.ShapeDtypeStruct((BATCH, EMBED), jnp.float32),
    compiler_params=pltpu.CompilerParams(use_tc_tiling_on_sc=False),
    scratch_types=[
        pltpu.VMEM((_NCHUNK, _CHUNK), jnp.int32),
        pltpu.VMEM((_B_PER_W, EMBED), jnp.float32),
        pltpu.SemaphoreType.DMA,
    ],
)
def _gather_sc(idx_hbm, tab_hbm, out_hbm, idx_v, rows_v, sem):
    wid = lax.axis_index("s") * _NC + lax.axis_index("c")
    base = wid * _B_PER_W

    pltpu.sync_copy(idx_hbm.at[wid], idx_v)
    copies = []
    for j in range(_NCHUNK):
        dst = rows_v.at[pl.ds(j * _CHUNK, _CHUNK)]
        copies.append(pltpu.async_copy(tab_hbm.at[idx_v.at[j]], dst, sem))
    for c in copies:
        c.wait()
    pltpu.sync_copy(rows_v, out_hbm.at[pl.ds(base, _B_PER_W)])


@functools.partial(
    pl.kernel,
    mesh=_mesh,
    out_type=jax.ShapeDtypeStruct((BATCH, EMBED), jnp.float32),
    compiler_params=pltpu.CompilerParams(use_tc_tiling_on_sc=False),
    scratch_types=[
        pltpu.VMEM((_NCHUNK, _CHUNK), jnp.int32),
        pltpu.VMEM((_B_PER_W, EMBED), jnp.float32),
        pltpu.VMEM((_B_PER_W, EMBED), jnp.float32),
        pltpu.SemaphoreType.DMA,
    ],
)
def _gather_mul_sc(idx_hbm, tab_hbm, other_hbm, out_hbm, idx_v, rows_v, oth_v, sem):
    wid = lax.axis_index("s") * _NC + lax.axis_index("c")
    base = wid * _B_PER_W

    pltpu.sync_copy(idx_hbm.at[wid], idx_v)
    copies = [pltpu.async_copy(other_hbm.at[pl.ds(base, _B_PER_W)], oth_v, sem)]
    for j in range(_NCHUNK):
        dst = rows_v.at[pl.ds(j * _CHUNK, _CHUNK)]
        copies.append(pltpu.async_copy(tab_hbm.at[idx_v.at[j]], dst, sem))
    for c in copies:
        c.wait()

    def row_body(r, _):
        for cbase in range(0, EMBED, LANES):
            sl = pl.ds(cbase, LANES)
            rows_v[r, sl] = rows_v[r, sl] * oth_v[r, sl]
        return 0

    lax.fori_loop(0, _B_PER_W, row_body, 0)
    pltpu.sync_copy(rows_v, out_hbm.at[pl.ds(base, _B_PER_W)])


def kernel(user_indices, item_indices, user_table, item_table):
    uidx = user_indices.astype(jnp.int32).reshape(_NW, _NCHUNK, _CHUNK)
    iidx = item_indices.astype(jnp.int32).reshape(_NW, _NCHUNK, _CHUNK)
    user_rows = _gather_sc(uidx, user_table)
    return _gather_mul_sc(iidx, item_table, user_rows)
